# baseline (speedup 1.0000x reference)
import functools

import jax
import jax.numpy as jnp
from jax import lax
from jax.experimental import pallas as pl
from jax.experimental.pallas import tpu as pltpu
from jax.experimental.pallas import tpu_sc as plsc

F32 = jnp.float32

N0 = 10000
N1 = 10000
E = 320000
EDGE_IN = 16
NODE_IN = 128
H = 64
FC = 128
C = 40

NC = 2
NS = 16
NW = NC * NS

NCH = 5
ECHUNK = E // NCH
ECH = 80
EW_C = ECHUNK // NW
ENC_C = EW_C // ECH

NPAD = 10240
SCH = 80
SNC = NPAD // (NW * SCH)

N1P = 10240
ROWS_PT = N1P // NS

BN0 = 2000
BE = 2000
BN = 2000


def _sigmoid(x):
    return 0.5 + 0.5 * jnp.tanh(0.5 * x)


def _ln_relu(x, g, b):
    m = jnp.mean(x, axis=-1, keepdims=True)
    xm = x - m
    v = jnp.mean(xm * xm, axis=-1, keepdims=True)
    return jnp.maximum(xm * lax.rsqrt(v + 1e-5) * g + b, 0.0)


def _nodepre_body(nf, hist, wa, wb, t_out, ps_out):
    pa = jnp.dot(nf[...], wa[...], preferred_element_type=F32)
    t_out[...] = jnp.concatenate([pa, hist[...]], axis=1)
    ps = jnp.dot(nf[...], wb[...], preferred_element_type=F32)
    ps_out[...] = jnp.concatenate([ps, ps], axis=1)


def _node_precompute(nf, hist, wsrc_t, wself_t):
    return pl.pallas_call(
        _nodepre_body,
        grid=(N0 // BN0,),
        in_specs=[
            pl.BlockSpec((BN0, NODE_IN), lambda i: (i, 0)),
            pl.BlockSpec((BN0, H), lambda i: (i, 0)),
            pl.BlockSpec((NODE_IN, H), lambda i: (0, 0)),
            pl.BlockSpec((NODE_IN, H), lambda i: (0, 0)),
        ],
        out_specs=[
            pl.BlockSpec((BN0, NODE_IN), lambda i: (i, 0)),
            pl.BlockSpec((BN0, NODE_IN), lambda i: (i, 0)),
        ],
        out_shape=[
            jax.ShapeDtypeStruct((N0, NODE_IN), F32),
            jax.ShapeDtypeStruct((N0, NODE_IN), F32),
        ],
    )(nf, hist, wsrc_t, wself_t)


def _gather_edges_body(t_hbm, esrc_hbm, g_hbm, eidx_v, erow0, erow1, sem0,
                       sem1):
    c = lax.axis_index("c")
    s = lax.axis_index("s")
    w = c * NS + s

    pltpu.sync_copy(esrc_hbm.at[w], eidx_v)

    def eloop(k, carry):
        j0 = 2 * k
        pltpu.async_copy(t_hbm.at[eidx_v.at[j0]], erow0, sem0).wait()
        cp1 = pltpu.async_copy(t_hbm.at[eidx_v.at[j0 + 1]], erow1, sem1)
        pltpu.sync_copy(erow0, g_hbm.at[pl.ds((w * ENC_C + j0) * ECH, ECH)])
        cp1.wait()
        pltpu.sync_copy(erow1,
                        g_hbm.at[pl.ds((w * ENC_C + j0 + 1) * ECH, ECH)])
        return carry

    lax.fori_loop(0, ENC_C // 2, eloop, 0)
    j = ENC_C - 1
    pltpu.async_copy(t_hbm.at[eidx_v.at[j]], erow0, sem0).wait()
    pltpu.sync_copy(erow0, g_hbm.at[pl.ds((w * ENC_C + j) * ECH, ECH)])


def _sc_gather_edges(table, esrc3):
    mesh = plsc.VectorSubcoreMesh(core_axis_name="c", subcore_axis_name="s",
                                  num_cores=NC, num_subcores=NS)
    return pl.kernel(
        _gather_edges_body,
        out_type=jax.ShapeDtypeStruct((ECHUNK, NODE_IN), F32),
        mesh=mesh,
        scratch_types=[
            pltpu.VMEM((ENC_C, ECH), jnp.int32),
            pltpu.VMEM((ECH, NODE_IN), F32),
            pltpu.VMEM((ECH, NODE_IN), F32),
            pltpu.SemaphoreType.DMA,
            pltpu.SemaphoreType.DMA,
        ],
    )(table, esrc3)


def _gather_self_body(ps_hbm, snid_hbm, s_hbm, sidx_v, srow_v, sem):
    c = lax.axis_index("c")
    s = lax.axis_index("s")
    w = c * NS + s

    pltpu.sync_copy(snid_hbm.at[w], sidx_v)

    def sloop(j, carry):
        pltpu.async_copy(ps_hbm.at[sidx_v.at[j]], srow_v, sem).wait()
        pltpu.sync_copy(srow_v, s_hbm.at[pl.ds((w * SNC + j) * SCH, SCH)])
        return carry

    lax.fori_loop(0, SNC, sloop, 0)


def _sc_gather_self(ps, snid3):
    mesh = plsc.VectorSubcoreMesh(core_axis_name="c", subcore_axis_name="s",
                                  num_cores=NC, num_subcores=NS)
    return pl.kernel(
        _gather_self_body,
        out_type=jax.ShapeDtypeStruct((NPAD, NODE_IN), F32),
        mesh=mesh,
        scratch_types=[
            pltpu.VMEM((SNC, SCH), jnp.int32),
            pltpu.VMEM((SCH, NODE_IN), F32),
            pltpu.SemaphoreType.DMA,
        ],
    )(ps, snid3)


def _edge_body(ef, g,
               wihr, wihz, wihn, whhr, whhz, whhn, we,
               brz_r, brz_z, bihn, bhhn, phib, phig, phibeta,
               out):
    efv = ef[...]
    x0v = efv[:, :EDGE_IN]
    x1v = efv[:, EDGE_IN:]
    r1 = _sigmoid(jnp.dot(x0v, wihr[...], preferred_element_type=F32)
                  + brz_r[...])
    z1 = _sigmoid(jnp.dot(x0v, wihz[...], preferred_element_type=F32)
                  + brz_z[...])
    n1 = jnp.tanh(jnp.dot(x0v, wihn[...], preferred_element_type=F32)
                  + bihn[...] + r1 * bhhn[...])
    h1 = (1.0 - z1) * n1

    r2 = _sigmoid(jnp.dot(x1v, wihr[...], preferred_element_type=F32)
                  + jnp.dot(h1, whhr[...], preferred_element_type=F32)
                  + brz_r[...])
    z2 = _sigmoid(jnp.dot(x1v, wihz[...], preferred_element_type=F32)
                  + jnp.dot(h1, whhz[...], preferred_element_type=F32)
                  + brz_z[...])
    n2 = jnp.tanh(jnp.dot(x1v, wihn[...], preferred_element_type=F32)
                  + bihn[...]
                  + r2 * (jnp.dot(h1, whhn[...], preferred_element_type=F32)
                          + bhhn[...]))
    h2 = (1.0 - z2) * n2 + z2 * h1

    emb = 0.5 * (h1 + h2)
    gv = g[...]
    pre = (jnp.dot(emb, we[...], preferred_element_type=F32)
           + gv[:, :H] + phib[...])
    nb = _ln_relu(pre, phig[...], phibeta[...])
    delta = jnp.maximum(nb - gv[:, H:], 0.0)
    out[...] = jnp.concatenate([delta, delta], axis=1)


def _edge_stage(ef, g, weights):
    wspecs = [pl.BlockSpec(w.shape, lambda i: tuple(0 for _ in w.shape))
              for w in weights]
    return pl.pallas_call(
        _edge_body,
        grid=(ECHUNK // BE,),
        in_specs=[
            pl.BlockSpec((BE, 2 * EDGE_IN), lambda i: (i, 0)),
            pl.BlockSpec((BE, NODE_IN), lambda i: (i, 0)),
        ] + wspecs,
        out_specs=pl.BlockSpec((BE, NODE_IN), lambda i: (i, 0)),
        out_shape=jax.ShapeDtypeStruct((ECHUNK, NODE_IN), F32),
    )(ef, g, *weights)


def _scatter_body(d_hbm, edst_hbm, z_hbm, p_hbm, acc_sh, idx_v, row_v):
    c = lax.axis_index("c")
    s = lax.axis_index("s")
    w = c * NS + s

    pltpu.sync_copy(z_hbm.at[pl.ds(s * ROWS_PT, ROWS_PT)],
                    acc_sh.at[pl.ds(s * ROWS_PT, ROWS_PT)])
    plsc.subcore_barrier()

    pltpu.sync_copy(edst_hbm.at[w], idx_v)

    def eloop(j, carry):
        pltpu.sync_copy(d_hbm.at[pl.ds((w * ENC_C + j) * ECH, ECH)], row_v)
        pltpu.sync_copy(row_v, acc_sh.at[idx_v.at[j]], add=True)
        return carry

    lax.fori_loop(0, ENC_C, eloop, 0)
    plsc.subcore_barrier()

    pltpu.sync_copy(acc_sh.at[pl.ds(s * ROWS_PT, ROWS_PT)],
                    p_hbm.at[c, pl.ds(s * ROWS_PT, ROWS_PT)])


def _sc_scatter(delta, edst3, zeros_init):
    mesh = plsc.VectorSubcoreMesh(core_axis_name="c", subcore_axis_name="s",
                                  num_cores=NC, num_subcores=NS)
    return pl.kernel(
        _scatter_body,
        out_type=jax.ShapeDtypeStruct((NC, N1P, NODE_IN), F32),
        mesh=mesh,
        scratch_types=[
            pltpu.VMEM_SHARED((N1P, NODE_IN), F32),
            pltpu.VMEM((ENC_C, ECH), jnp.int32),
            pltpu.VMEM((ECH, NODE_IN), F32),
        ],
    )(delta, edst3, zeros_init)


def _out_body(p0, p1, p2, p3, p4, sg, subg,
              phib, phig, phibeta, owa, owb, outb, outg, outbeta,
              fc1w, fc1b, fc2w, fc2b,
              out):
    delta_nb = (p0[0, :, :H] + p0[1, :, :H]
                + p1[0, :, :H] + p1[1, :, :H]
                + p2[0, :, :H] + p2[1, :, :H]
                + p3[0, :, :H] + p3[1, :, :H]
                + p4[0, :, :H] + p4[1, :, :H])
    self_h = _ln_relu(sg[:, :H] + phib[...], phig[...], phibeta[...])
    a = (delta_nb - self_h) * subg[...]
    pre = (jnp.dot(a, owa[...], preferred_element_type=F32)
           + jnp.dot(self_h, owb[...], preferred_element_type=F32)
           + outb[...])
    new_h = _ln_relu(pre, outg[...], outbeta[...])
    hf = jnp.dot(new_h, fc1w[...], preferred_element_type=F32) + fc1b[...]
    out[...] = jnp.dot(hf, fc2w[...], preferred_element_type=F32) + fc2b[...]


def _out_stage(partials, sgath, subg, weights):
    wspecs = [pl.BlockSpec(w.shape, lambda i: tuple(0 for _ in w.shape))
              for w in weights]
    pspecs = [pl.BlockSpec((NC, BN, NODE_IN), lambda i: (0, i, 0))
              for _ in partials]
    return pl.pallas_call(
        _out_body,
        grid=(N1 // BN,),
        in_specs=pspecs + [
            pl.BlockSpec((BN, NODE_IN), lambda i: (i, 0)),
            pl.BlockSpec((BN, 1), lambda i: (i, 0)),
        ] + wspecs,
        out_specs=pl.BlockSpec((BN, C), lambda i: (i, 0)),
        out_shape=jax.ShapeDtypeStruct((N1, C), F32),
    )(*partials, sgath, subg, *weights)


def kernel(node_features, edge_features, history_0, subg_norm,
           self_layer_nid, edge_src, edge_dst,
           gru_Wih, gru_Whh, gru_bih, gru_bhh,
           phi_W, phi_b, phi_g, phi_beta,
           out_W, out_b, out_g, out_beta,
           fc1_W, fc1_b, fc2_W, fc2_b):
    wsrc_t = phi_W[:, :NODE_IN].T
    wself_t = phi_W[:, H:].T
    we_t = phi_W[:, NODE_IN:].T
    wihr = gru_Wih[:H, :].T
    wihz = gru_Wih[H:2 * H, :].T
    wihn = gru_Wih[2 * H:, :].T
    whhr = gru_Whh[:H, :].T
    whhz = gru_Whh[H:2 * H, :].T
    whhn = gru_Whh[2 * H:, :].T
    brz_r = (gru_bih[:H] + gru_bhh[:H]).reshape(1, H)
    brz_z = (gru_bih[H:2 * H] + gru_bhh[H:2 * H]).reshape(1, H)
    bihn = gru_bih[2 * H:].reshape(1, H)
    bhhn = gru_bhh[2 * H:].reshape(1, H)
    phib = phi_b.reshape(1, H)
    phig = phi_g.reshape(1, H)
    phibeta = phi_beta.reshape(1, H)
    owa = out_W[:, :H].T
    owb = out_W[:, H:].T
    outb = out_b.reshape(1, H)
    outg = out_g.reshape(1, H)
    outbeta = out_beta.reshape(1, H)
    fc1t = fc1_W.T
    fc1b = fc1_b.reshape(1, FC)
    fc2t = fc2_W.T
    fc2b = fc2_b.reshape(1, C)

    esrc4 = edge_src.reshape(NCH, NW, ENC_C, ECH)
    edst4 = edge_dst.reshape(NCH, NW, ENC_C, ECH)
    snid_pad = jnp.concatenate(
        [self_layer_nid, jnp.zeros((NPAD - N1,), jnp.int32)]
    ).reshape(NW, SNC, SCH)
    ef3 = edge_features.reshape(NCH, ECHUNK, 2 * EDGE_IN)
    zeros_init = jnp.zeros((N1P, NODE_IN), F32)

    edge_w = [
        wihr, wihz, wihn, whhr, whhz, whhn, we_t,
        brz_r, brz_z, bihn, bhhn, phib, phig, phibeta,
    ]

    table, ps = _node_precompute(node_features, history_0, wsrc_t, wself_t)
    s_pad = _sc_gather_self(ps, snid_pad)
    partials = []
    for chk in range(NCH):
        g = _sc_gather_edges(table, esrc4[chk])
        delta = _edge_stage(ef3[chk], g, edge_w)
        partials.append(_sc_scatter(delta, edst4[chk], zeros_init))
    logit = _out_stage(partials, s_pad[:N1], subg_norm, [
        phib, phig, phibeta, owa, owb, outb, outg, outbeta,
        fc1t, fc1b, fc2t, fc2b,
    ])
    return logit


# speedup vs baseline: 1.2604x; 1.2604x over previous
import functools

import jax
import jax.numpy as jnp
from jax import lax
from jax.experimental import pallas as pl
from jax.experimental.pallas import tpu as pltpu
from jax.experimental.pallas import tpu_sc as plsc

F32 = jnp.float32

N0 = 10000
N1 = 10000
E = 320000
EDGE_IN = 16
NODE_IN = 128
H = 64
FC = 128
C = 40

NC = 2
NS = 16
NW = NC * NS

NCH = 5
ECHUNK = E // NCH
ECH = 80
EW_C = ECHUNK // NW
ENC_C = EW_C // ECH

NPAD = 10240
SCH = 80
SNC = NPAD // (NW * SCH)

N1P = 10240
ROWS_PT = N1P // NS

BN0 = 2000
BE = 2000
BN = 2000


def _sigmoid(x):
    return 0.5 + 0.5 * jnp.tanh(0.5 * x)


def _ln_relu(x, g, b):
    m = jnp.mean(x, axis=-1, keepdims=True)
    xm = x - m
    v = jnp.mean(xm * xm, axis=-1, keepdims=True)
    return jnp.maximum(xm * lax.rsqrt(v + 1e-5) * g + b, 0.0)


def _nodepre_body(nf, hist, wa, wb, t_out, ps_out):
    pa = jnp.dot(nf[...], wa[...], preferred_element_type=F32)
    t_out[...] = jnp.concatenate([pa, hist[...]], axis=1)
    ps = jnp.dot(nf[...], wb[...], preferred_element_type=F32)
    ps_out[...] = jnp.concatenate([ps, ps], axis=1)


def _node_precompute(nf, hist, wsrc_t, wself_t):
    return pl.pallas_call(
        _nodepre_body,
        grid=(N0 // BN0,),
        in_specs=[
            pl.BlockSpec((BN0, NODE_IN), lambda i: (i, 0)),
            pl.BlockSpec((BN0, H), lambda i: (i, 0)),
            pl.BlockSpec((NODE_IN, H), lambda i: (0, 0)),
            pl.BlockSpec((NODE_IN, H), lambda i: (0, 0)),
        ],
        out_specs=[
            pl.BlockSpec((BN0, NODE_IN), lambda i: (i, 0)),
            pl.BlockSpec((BN0, NODE_IN), lambda i: (i, 0)),
        ],
        out_shape=[
            jax.ShapeDtypeStruct((N0, NODE_IN), F32),
            jax.ShapeDtypeStruct((N0, NODE_IN), F32),
        ],
    )(nf, hist, wsrc_t, wself_t)


def _gather_edges_body(chk, t_hbm, esrc_hbm, g_hbm, eidx_v, erow0, erow1,
                       sem0, sem1):
    c = lax.axis_index("c")
    s = lax.axis_index("s")
    w = c * NS + s

    pltpu.sync_copy(esrc_hbm.at[chk * NW + w], eidx_v)

    def eloop(k, carry):
        j0 = 2 * k
        pltpu.async_copy(t_hbm.at[eidx_v.at[j0]], erow0, sem0).wait()
        cp1 = pltpu.async_copy(t_hbm.at[eidx_v.at[j0 + 1]], erow1, sem1)
        pltpu.sync_copy(erow0, g_hbm.at[pl.ds((w * ENC_C + j0) * ECH, ECH)])
        cp1.wait()
        pltpu.sync_copy(erow1,
                        g_hbm.at[pl.ds((w * ENC_C + j0 + 1) * ECH, ECH)])
        return carry

    lax.fori_loop(0, ENC_C // 2, eloop, 0)
    j = ENC_C - 1
    pltpu.async_copy(t_hbm.at[eidx_v.at[j]], erow0, sem0).wait()
    pltpu.sync_copy(erow0, g_hbm.at[pl.ds((w * ENC_C + j) * ECH, ECH)])


def _sc_gather_edges(table, esrc3, chk):
    mesh = plsc.VectorSubcoreMesh(core_axis_name="c", subcore_axis_name="s",
                                  num_cores=NC, num_subcores=NS)
    return pl.kernel(
        functools.partial(_gather_edges_body, chk),
        out_type=jax.ShapeDtypeStruct((ECHUNK, NODE_IN), F32),
        mesh=mesh,
        scratch_types=[
            pltpu.VMEM((ENC_C, ECH), jnp.int32),
            pltpu.VMEM((ECH, NODE_IN), F32),
            pltpu.VMEM((ECH, NODE_IN), F32),
            pltpu.SemaphoreType.DMA,
            pltpu.SemaphoreType.DMA,
        ],
        name=f"gather_edges_c{chk}",
    )(table, esrc3)


def _gather_self_body(ps_hbm, snid_hbm, s_hbm, sidx_v, srow_v, sem):
    c = lax.axis_index("c")
    s = lax.axis_index("s")
    w = c * NS + s

    pltpu.sync_copy(snid_hbm.at[w], sidx_v)

    def sloop(j, carry):
        pltpu.async_copy(ps_hbm.at[sidx_v.at[j]], srow_v, sem).wait()
        pltpu.sync_copy(srow_v, s_hbm.at[pl.ds((w * SNC + j) * SCH, SCH)])
        return carry

    lax.fori_loop(0, SNC, sloop, 0)


def _sc_gather_self(ps, snid3):
    mesh = plsc.VectorSubcoreMesh(core_axis_name="c", subcore_axis_name="s",
                                  num_cores=NC, num_subcores=NS)
    return pl.kernel(
        _gather_self_body,
        out_type=jax.ShapeDtypeStruct((NPAD, NODE_IN), F32),
        mesh=mesh,
        scratch_types=[
            pltpu.VMEM((SNC, SCH), jnp.int32),
            pltpu.VMEM((SCH, NODE_IN), F32),
            pltpu.SemaphoreType.DMA,
        ],
    )(ps, snid3)


def _edge_body(ef, g,
               wihr, wihz, wihn, whhr, whhz, whhn, we,
               brz_r, brz_z, bihn, bhhn, phib, phig, phibeta,
               out):
    efv = ef[...]
    x0v = efv[:, :EDGE_IN]
    x1v = efv[:, EDGE_IN:]
    r1 = _sigmoid(jnp.dot(x0v, wihr[...], preferred_element_type=F32)
                  + brz_r[...])
    z1 = _sigmoid(jnp.dot(x0v, wihz[...], preferred_element_type=F32)
                  + brz_z[...])
    n1 = jnp.tanh(jnp.dot(x0v, wihn[...], preferred_element_type=F32)
                  + bihn[...] + r1 * bhhn[...])
    h1 = (1.0 - z1) * n1

    r2 = _sigmoid(jnp.dot(x1v, wihr[...], preferred_element_type=F32)
                  + jnp.dot(h1, whhr[...], preferred_element_type=F32)
                  + brz_r[...])
    z2 = _sigmoid(jnp.dot(x1v, wihz[...], preferred_element_type=F32)
                  + jnp.dot(h1, whhz[...], preferred_element_type=F32)
                  + brz_z[...])
    n2 = jnp.tanh(jnp.dot(x1v, wihn[...], preferred_element_type=F32)
                  + bihn[...]
                  + r2 * (jnp.dot(h1, whhn[...], preferred_element_type=F32)
                          + bhhn[...]))
    h2 = (1.0 - z2) * n2 + z2 * h1

    emb = 0.5 * (h1 + h2)
    gv = g[...]
    pre = (jnp.dot(emb, we[...], preferred_element_type=F32)
           + gv[:, :H] + phib[...])
    nb = _ln_relu(pre, phig[...], phibeta[...])
    delta = jnp.maximum(nb - gv[:, H:], 0.0)
    out[...] = jnp.concatenate([delta, delta], axis=1)


def _edge_stage(ef2, g, chk, weights):
    nblk = ECHUNK // BE
    base = chk * nblk
    wspecs = [pl.BlockSpec(w.shape, lambda i: tuple(0 for _ in w.shape))
              for w in weights]
    return pl.pallas_call(
        _edge_body,
        grid=(nblk,),
        in_specs=[
            pl.BlockSpec((BE, 2 * EDGE_IN), lambda i: (base + i, 0)),
            pl.BlockSpec((BE, NODE_IN), lambda i: (i, 0)),
        ] + wspecs,
        out_specs=pl.BlockSpec((BE, NODE_IN), lambda i: (i, 0)),
        out_shape=jax.ShapeDtypeStruct((ECHUNK, NODE_IN), F32),
    )(ef2, g, *weights)


def _scatter_body(chk, d_hbm, edst_hbm, p_hbm, acc_sh, idx_v, row_v):
    c = lax.axis_index("c")
    s = lax.axis_index("s")
    w = c * NS + s

    zero16 = jnp.zeros((16,), F32)

    def zrow(r, carry):
        def zcol(k, c2):
            row_v[r, pl.ds(k * 16, 16)] = zero16
            return c2
        return lax.fori_loop(0, NODE_IN // 16, zcol, carry)

    lax.fori_loop(0, ECH, zrow, 0)
    for k in range(ROWS_PT // ECH):
        pltpu.sync_copy(row_v,
                        acc_sh.at[pl.ds(s * ROWS_PT + k * ECH, ECH)])
    plsc.subcore_barrier()

    pltpu.sync_copy(edst_hbm.at[chk * NW + w], idx_v)

    def eloop(j, carry):
        pltpu.sync_copy(d_hbm.at[pl.ds((w * ENC_C + j) * ECH, ECH)], row_v)
        pltpu.sync_copy(row_v, acc_sh.at[idx_v.at[j]], add=True)
        return carry

    lax.fori_loop(0, ENC_C, eloop, 0)
    plsc.subcore_barrier()

    pltpu.sync_copy(acc_sh.at[pl.ds(s * ROWS_PT, ROWS_PT)],
                    p_hbm.at[c, pl.ds(s * ROWS_PT, ROWS_PT)])


def _sc_scatter(delta, edst3, chk):
    mesh = plsc.VectorSubcoreMesh(core_axis_name="c", subcore_axis_name="s",
                                  num_cores=NC, num_subcores=NS)
    return pl.kernel(
        functools.partial(_scatter_body, chk),
        out_type=jax.ShapeDtypeStruct((NC, N1P, NODE_IN), F32),
        mesh=mesh,
        scratch_types=[
            pltpu.VMEM_SHARED((N1P, NODE_IN), F32),
            pltpu.VMEM((ENC_C, ECH), jnp.int32),
            pltpu.VMEM((ECH, NODE_IN), F32),
        ],
        name=f"scatter_c{chk}",
    )(delta, edst3)


def _out_body(p0, p1, p2, p3, p4, sg, subg,
              phib, phig, phibeta, owa, owb, outb, outg, outbeta,
              fc1w, fc1b, fc2w, fc2b,
              out):
    delta_nb = (p0[0, :, :H] + p0[1, :, :H]
                + p1[0, :, :H] + p1[1, :, :H]
                + p2[0, :, :H] + p2[1, :, :H]
                + p3[0, :, :H] + p3[1, :, :H]
                + p4[0, :, :H] + p4[1, :, :H])
    self_h = _ln_relu(sg[:, :H] + phib[...], phig[...], phibeta[...])
    a = (delta_nb - self_h) * subg[...]
    pre = (jnp.dot(a, owa[...], preferred_element_type=F32)
           + jnp.dot(self_h, owb[...], preferred_element_type=F32)
           + outb[...])
    new_h = _ln_relu(pre, outg[...], outbeta[...])
    hf = jnp.dot(new_h, fc1w[...], preferred_element_type=F32) + fc1b[...]
    out[...] = jnp.dot(hf, fc2w[...], preferred_element_type=F32) + fc2b[...]


def _out_stage(partials, sgath, subg, weights):
    wspecs = [pl.BlockSpec(w.shape, lambda i: tuple(0 for _ in w.shape))
              for w in weights]
    pspecs = [pl.BlockSpec((NC, BN, NODE_IN), lambda i: (0, i, 0))
              for _ in partials]
    return pl.pallas_call(
        _out_body,
        grid=(N1 // BN,),
        in_specs=pspecs + [
            pl.BlockSpec((BN, NODE_IN), lambda i: (i, 0)),
            pl.BlockSpec((BN, 1), lambda i: (i, 0)),
        ] + wspecs,
        out_specs=pl.BlockSpec((BN, C), lambda i: (i, 0)),
        out_shape=jax.ShapeDtypeStruct((N1, C), F32),
    )(*partials, sgath, subg, *weights)


def kernel(node_features, edge_features, history_0, subg_norm,
           self_layer_nid, edge_src, edge_dst,
           gru_Wih, gru_Whh, gru_bih, gru_bhh,
           phi_W, phi_b, phi_g, phi_beta,
           out_W, out_b, out_g, out_beta,
           fc1_W, fc1_b, fc2_W, fc2_b):
    wsrc_t = phi_W[:, :NODE_IN].T
    wself_t = phi_W[:, H:].T
    we_t = phi_W[:, NODE_IN:].T
    wihr = gru_Wih[:H, :].T
    wihz = gru_Wih[H:2 * H, :].T
    wihn = gru_Wih[2 * H:, :].T
    whhr = gru_Whh[:H, :].T
    whhz = gru_Whh[H:2 * H, :].T
    whhn = gru_Whh[2 * H:, :].T
    brz_r = (gru_bih[:H] + gru_bhh[:H]).reshape(1, H)
    brz_z = (gru_bih[H:2 * H] + gru_bhh[H:2 * H]).reshape(1, H)
    bihn = gru_bih[2 * H:].reshape(1, H)
    bhhn = gru_bhh[2 * H:].reshape(1, H)
    phib = phi_b.reshape(1, H)
    phig = phi_g.reshape(1, H)
    phibeta = phi_beta.reshape(1, H)
    owa = out_W[:, :H].T
    owb = out_W[:, H:].T
    outb = out_b.reshape(1, H)
    outg = out_g.reshape(1, H)
    outbeta = out_beta.reshape(1, H)
    fc1t = fc1_W.T
    fc1b = fc1_b.reshape(1, FC)
    fc2t = fc2_W.T
    fc2b = fc2_b.reshape(1, C)

    esrc3 = edge_src.reshape(NCH * NW, ENC_C, ECH)
    edst3 = edge_dst.reshape(NCH * NW, ENC_C, ECH)
    snid_pad = jnp.concatenate(
        [self_layer_nid, jnp.zeros((NPAD - N1,), jnp.int32)]
    ).reshape(NW, SNC, SCH)
    ef2 = edge_features.reshape(E, 2 * EDGE_IN)

    edge_w = [
        wihr, wihz, wihn, whhr, whhz, whhn, we_t,
        brz_r, brz_z, bihn, bhhn, phib, phig, phibeta,
    ]

    table, ps = _node_precompute(node_features, history_0, wsrc_t, wself_t)
    s_pad = _sc_gather_self(ps, snid_pad)
    partials = []
    for chk in range(NCH):
        g = _sc_gather_edges(table, esrc3, chk)
        delta = _edge_stage(ef2, g, chk, edge_w)
        partials.append(_sc_scatter(delta, edst3, chk))
    logit = _out_stage(partials, s_pad[:N1], subg_norm, [
        phib, phig, phibeta, owa, owb, outb, outg, outbeta,
        fc1t, fc1b, fc2t, fc2b,
    ])
    return logit


# speedup vs baseline: 1.9104x; 1.5157x over previous
import functools

import jax
import jax.numpy as jnp
from jax import lax
from jax.experimental import pallas as pl
from jax.experimental.pallas import tpu as pltpu
from jax.experimental.pallas import tpu_sc as plsc

F32 = jnp.float32

N0 = 10000
N1 = 10000
E = 320000
EDGE_IN = 16
NODE_IN = 128
H = 64
FC = 128
C = 40

NC = 2
NS = 16
NW = NC * NS

NCH = 5
ECHUNK = E // NCH
ECH = 80
EW_C = ECHUNK // NW
ENC_C = EW_C // ECH

NPAD = 10240
SCH = 80
SNC = NPAD // (NW * SCH)

N1P = 10240
ROWS_PT = N1P // NS

BN0 = 2000
BE = 2000
BN = 2000


def _sigmoid(x):
    return 0.5 + 0.5 * jnp.tanh(0.5 * x)


def _ln_relu(x, g, b):
    m = jnp.mean(x, axis=-1, keepdims=True)
    xm = x - m
    v = jnp.mean(xm * xm, axis=-1, keepdims=True)
    return jnp.maximum(xm * lax.rsqrt(v + 1e-5) * g + b, 0.0)


def _nodepre_body(nf, hist, wa, wb, t_out, ps_out):
    pa = jnp.dot(nf[...], wa[...], preferred_element_type=F32)
    t_out[...] = jnp.concatenate([pa, hist[...]], axis=1)
    ps = jnp.dot(nf[...], wb[...], preferred_element_type=F32)
    ps_out[...] = jnp.concatenate([ps, ps], axis=1)


def _node_precompute(nf, hist, wsrc_t, wself_t):
    return pl.pallas_call(
        _nodepre_body,
        grid=(N0 // BN0,),
        in_specs=[
            pl.BlockSpec((BN0, NODE_IN), lambda i: (i, 0)),
            pl.BlockSpec((BN0, H), lambda i: (i, 0)),
            pl.BlockSpec((NODE_IN, H), lambda i: (0, 0)),
            pl.BlockSpec((NODE_IN, H), lambda i: (0, 0)),
        ],
        out_specs=[
            pl.BlockSpec((BN0, NODE_IN), lambda i: (i, 0)),
            pl.BlockSpec((BN0, NODE_IN), lambda i: (i, 0)),
        ],
        out_shape=[
            jax.ShapeDtypeStruct((N0, NODE_IN), F32),
            jax.ShapeDtypeStruct((N0, NODE_IN), F32),
        ],
    )(nf, hist, wsrc_t, wself_t)


def _gather_edges_body(chk, t_hbm, esrc_hbm, g_hbm, eidx_v, erow0, erow1,
                       sem0, sem1):
    c = lax.axis_index("c")
    s = lax.axis_index("s")
    w = c * NS + s

    pltpu.sync_copy(esrc_hbm.at[chk * NW + w], eidx_v)

    def eloop(k, carry):
        j0 = 2 * k
        pltpu.async_copy(t_hbm.at[eidx_v.at[j0]], erow0, sem0).wait()
        cp1 = pltpu.async_copy(t_hbm.at[eidx_v.at[j0 + 1]], erow1, sem1)
        pltpu.sync_copy(erow0, g_hbm.at[pl.ds((w * ENC_C + j0) * ECH, ECH)])
        cp1.wait()
        pltpu.sync_copy(erow1,
                        g_hbm.at[pl.ds((w * ENC_C + j0 + 1) * ECH, ECH)])
        return carry

    lax.fori_loop(0, ENC_C // 2, eloop, 0)
    j = ENC_C - 1
    pltpu.async_copy(t_hbm.at[eidx_v.at[j]], erow0, sem0).wait()
    pltpu.sync_copy(erow0, g_hbm.at[pl.ds((w * ENC_C + j) * ECH, ECH)])


def _sc_gather_edges(table, esrc3, chk):
    mesh = plsc.VectorSubcoreMesh(core_axis_name="c", subcore_axis_name="s",
                                  num_cores=NC, num_subcores=NS)
    return pl.kernel(
        functools.partial(_gather_edges_body, chk),
        out_type=jax.ShapeDtypeStruct((ECHUNK, NODE_IN), F32),
        mesh=mesh,
        scratch_types=[
            pltpu.VMEM((ENC_C, ECH), jnp.int32),
            pltpu.VMEM((ECH, NODE_IN), F32),
            pltpu.VMEM((ECH, NODE_IN), F32),
            pltpu.SemaphoreType.DMA,
            pltpu.SemaphoreType.DMA,
        ],
        name=f"gather_edges_c{chk}",
    )(table, esrc3)


def _gather_self_body(ps_hbm, snid_hbm, s_hbm, sidx_v, srow_v, sem):
    c = lax.axis_index("c")
    s = lax.axis_index("s")
    w = c * NS + s

    pltpu.sync_copy(snid_hbm.at[w], sidx_v)

    def sloop(j, carry):
        pltpu.async_copy(ps_hbm.at[sidx_v.at[j]], srow_v, sem).wait()
        pltpu.sync_copy(srow_v, s_hbm.at[pl.ds((w * SNC + j) * SCH, SCH)])
        return carry

    lax.fori_loop(0, SNC, sloop, 0)


def _sc_gather_self(ps, snid3):
    mesh = plsc.VectorSubcoreMesh(core_axis_name="c", subcore_axis_name="s",
                                  num_cores=NC, num_subcores=NS)
    return pl.kernel(
        _gather_self_body,
        out_type=jax.ShapeDtypeStruct((NPAD, NODE_IN), F32),
        mesh=mesh,
        scratch_types=[
            pltpu.VMEM((SNC, SCH), jnp.int32),
            pltpu.VMEM((SCH, NODE_IN), F32),
            pltpu.SemaphoreType.DMA,
        ],
    )(ps, snid3)


def _edge_body(ef_lo, ef_hi, g_lo, g_hi,
               wih_r, wih_z, wih_n, whh_r, whh_z, whh_n, we2, mean2,
               brz_r, brz_z, bihn, bhhn, phib, phig, phibeta,
               out):
    elo = ef_lo[...]
    ehi = ef_hi[...]
    x0 = jnp.concatenate([elo[:, :EDGE_IN], ehi[:, :EDGE_IN]], axis=1)
    x1 = jnp.concatenate([elo[:, EDGE_IN:], ehi[:, EDGE_IN:]], axis=1)
    glo = g_lo[...]
    ghi = g_hi[...]
    gsrc = jnp.concatenate([glo[:, :H], ghi[:, :H]], axis=1)
    hist = jnp.concatenate([glo[:, H:], ghi[:, H:]], axis=1)

    r1 = _sigmoid(jnp.dot(x0, wih_r[...], preferred_element_type=F32)
                  + brz_r[...])
    z1 = _sigmoid(jnp.dot(x0, wih_z[...], preferred_element_type=F32)
                  + brz_z[...])
    n1 = jnp.tanh(jnp.dot(x0, wih_n[...], preferred_element_type=F32)
                  + bihn[...] + r1 * bhhn[...])
    h1 = (1.0 - z1) * n1

    r2 = _sigmoid(jnp.dot(x1, wih_r[...], preferred_element_type=F32)
                  + jnp.dot(h1, whh_r[...], preferred_element_type=F32)
                  + brz_r[...])
    z2 = _sigmoid(jnp.dot(x1, wih_z[...], preferred_element_type=F32)
                  + jnp.dot(h1, whh_z[...], preferred_element_type=F32)
                  + brz_z[...])
    n2 = jnp.tanh(jnp.dot(x1, wih_n[...], preferred_element_type=F32)
                  + bihn[...]
                  + r2 * (jnp.dot(h1, whh_n[...], preferred_element_type=F32)
                          + bhhn[...]))
    h2 = (1.0 - z2) * n2 + z2 * h1

    emb = 0.5 * (h1 + h2)
    pre = (jnp.dot(emb, we2[...], preferred_element_type=F32)
           + gsrc + phib[...])
    m = jnp.dot(pre, mean2[...], preferred_element_type=F32)
    q = jnp.dot(pre * pre, mean2[...], preferred_element_type=F32)
    v = q - m * m
    nb = jnp.maximum((pre - m) * lax.rsqrt(v + 1e-5) * phig[...]
                     + phibeta[...], 0.0)
    out[...] = jnp.maximum(nb - hist, 0.0)


def _edge_stage(ef2, g, chk, weights):
    nhb = ECHUNK // (2 * BE)
    base = chk * (ECHUNK // BE)
    wspecs = [pl.BlockSpec(w.shape, lambda i: tuple(0 for _ in w.shape))
              for w in weights]
    return pl.pallas_call(
        _edge_body,
        grid=(nhb,),
        in_specs=[
            pl.BlockSpec((BE, 2 * EDGE_IN), lambda i: (base + i, 0)),
            pl.BlockSpec((BE, 2 * EDGE_IN), lambda i: (base + nhb + i, 0)),
            pl.BlockSpec((BE, NODE_IN), lambda i: (i, 0)),
            pl.BlockSpec((BE, NODE_IN), lambda i: (nhb + i, 0)),
        ] + wspecs,
        out_specs=pl.BlockSpec((BE, NODE_IN), lambda i: (i, 0)),
        out_shape=jax.ShapeDtypeStruct((ECHUNK // 2, NODE_IN), F32),
    )(ef2, ef2, g, g, *weights)


def _scatter_body(chk, d_hbm, edst_hbm, p_hbm, acc_sh, idx_v, row_v):
    c = lax.axis_index("c")
    s = lax.axis_index("s")

    zero16 = jnp.zeros((16,), F32)

    def zrow(r, carry):
        def zcol(k, c2):
            row_v[r, pl.ds(k * 16, 16)] = zero16
            return c2
        return lax.fori_loop(0, NODE_IN // 16, zcol, carry)

    lax.fori_loop(0, ECH, zrow, 0)
    for k in range(ROWS_PT // ECH):
        pltpu.sync_copy(row_v,
                        acc_sh.at[pl.ds(s * ROWS_PT + k * ECH, ECH)])
    plsc.subcore_barrier()

    pltpu.sync_copy(edst_hbm.at[(chk * NC + c) * NS + s], idx_v)

    def eloop(j, carry):
        pltpu.sync_copy(d_hbm.at[pl.ds((s * ENC_C + j) * ECH, ECH)], row_v)
        pltpu.sync_copy(row_v, acc_sh.at[idx_v.at[j]], add=True)
        return carry

    lax.fori_loop(0, ENC_C, eloop, 0)
    plsc.subcore_barrier()

    pltpu.sync_copy(acc_sh.at[pl.ds(s * ROWS_PT, ROWS_PT)],
                    p_hbm.at[c, pl.ds(s * ROWS_PT, ROWS_PT)])


def _sc_scatter(delta, edst3, chk):
    mesh = plsc.VectorSubcoreMesh(core_axis_name="c", subcore_axis_name="s",
                                  num_cores=NC, num_subcores=NS)
    return pl.kernel(
        functools.partial(_scatter_body, chk),
        out_type=jax.ShapeDtypeStruct((NC, N1P, NODE_IN), F32),
        mesh=mesh,
        scratch_types=[
            pltpu.VMEM_SHARED((N1P, NODE_IN), F32),
            pltpu.VMEM((ENC_C, ECH), jnp.int32),
            pltpu.VMEM((ECH, NODE_IN), F32),
        ],
        name=f"scatter_c{chk}",
    )(delta, edst3)


def _out_body(p0, p1, p2, p3, p4, sg, subg,
              phib, phig, phibeta, owa, owb, outb, outg, outbeta,
              fc1w, fc1b, fc2w, fc2b,
              out):
    delta_nb = (p0[0, :, :H] + p0[1, :, H:]
                + p1[0, :, :H] + p1[1, :, H:]
                + p2[0, :, :H] + p2[1, :, H:]
                + p3[0, :, :H] + p3[1, :, H:]
                + p4[0, :, :H] + p4[1, :, H:])
    self_h = _ln_relu(sg[:, :H] + phib[...], phig[...], phibeta[...])
    a = (delta_nb - self_h) * subg[...]
    pre = (jnp.dot(a, owa[...], preferred_element_type=F32)
           + jnp.dot(self_h, owb[...], preferred_element_type=F32)
           + outb[...])
    new_h = _ln_relu(pre, outg[...], outbeta[...])
    hf = jnp.dot(new_h, fc1w[...], preferred_element_type=F32) + fc1b[...]
    out[...] = jnp.dot(hf, fc2w[...], preferred_element_type=F32) + fc2b[...]


def _out_stage(partials, sgath, subg, weights):
    wspecs = [pl.BlockSpec(w.shape, lambda i: tuple(0 for _ in w.shape))
              for w in weights]
    pspecs = [pl.BlockSpec((NC, BN, NODE_IN), lambda i: (0, i, 0))
              for _ in partials]
    return pl.pallas_call(
        _out_body,
        grid=(N1 // BN,),
        in_specs=pspecs + [
            pl.BlockSpec((BN, NODE_IN), lambda i: (i, 0)),
            pl.BlockSpec((BN, 1), lambda i: (i, 0)),
        ] + wspecs,
        out_specs=pl.BlockSpec((BN, C), lambda i: (i, 0)),
        out_shape=jax.ShapeDtypeStruct((N1, C), F32),
    )(*partials, sgath, subg, *weights)


def kernel(node_features, edge_features, history_0, subg_norm,
           self_layer_nid, edge_src, edge_dst,
           gru_Wih, gru_Whh, gru_bih, gru_bhh,
           phi_W, phi_b, phi_g, phi_beta,
           out_W, out_b, out_g, out_beta,
           fc1_W, fc1_b, fc2_W, fc2_b):
    wsrc_t = phi_W[:, :NODE_IN].T
    wself_t = phi_W[:, H:].T
    we_t = phi_W[:, NODE_IN:].T
    wihr = gru_Wih[:H, :].T
    wihz = gru_Wih[H:2 * H, :].T
    wihn = gru_Wih[2 * H:, :].T
    whhr = gru_Whh[:H, :].T
    whhz = gru_Whh[H:2 * H, :].T
    whhn = gru_Whh[2 * H:, :].T
    brz_r = (gru_bih[:H] + gru_bhh[:H]).reshape(1, H)
    brz_z = (gru_bih[H:2 * H] + gru_bhh[H:2 * H]).reshape(1, H)
    bihn = gru_bih[2 * H:].reshape(1, H)
    bhhn = gru_bhh[2 * H:].reshape(1, H)
    phib = phi_b.reshape(1, H)
    phig = phi_g.reshape(1, H)
    phibeta = phi_beta.reshape(1, H)
    owa = out_W[:, :H].T
    owb = out_W[:, H:].T
    outb = out_b.reshape(1, H)
    outg = out_g.reshape(1, H)
    outbeta = out_beta.reshape(1, H)
    fc1t = fc1_W.T
    fc1b = fc1_b.reshape(1, FC)
    fc2t = fc2_W.T
    fc2b = fc2_b.reshape(1, C)

    def _bd(wm):
        k = wm.shape[0]
        z = jnp.zeros((k, H), F32)
        return jnp.concatenate([
            jnp.concatenate([wm, z], axis=1),
            jnp.concatenate([z, wm], axis=1),
        ], axis=0)

    def _t2(b):
        return jnp.concatenate([b, b], axis=1)

    wih_r2, wih_z2, wih_n2 = _bd(wihr), _bd(wihz), _bd(wihn)
    whh_r2, whh_z2, whh_n2 = _bd(whhr), _bd(whhz), _bd(whhn)
    we2 = _bd(we_t)
    mean2 = _bd(jnp.full((H, H), 1.0 / H, F32))

    esrc3 = edge_src.reshape(NCH * NW, ENC_C, ECH)
    edst3 = edge_dst.reshape(NCH * NW, ENC_C, ECH)
    snid_pad = jnp.concatenate(
        [self_layer_nid, jnp.zeros((NPAD - N1,), jnp.int32)]
    ).reshape(NW, SNC, SCH)
    ef2 = edge_features.reshape(E, 2 * EDGE_IN)

    edge_w = [
        wih_r2, wih_z2, wih_n2, whh_r2, whh_z2, whh_n2, we2, mean2,
        _t2(brz_r), _t2(brz_z), _t2(bihn), _t2(bhhn),
        _t2(phib), _t2(phig), _t2(phibeta),
    ]

    table, ps = _node_precompute(node_features, history_0, wsrc_t, wself_t)
    s_pad = _sc_gather_self(ps, snid_pad)
    partials = []
    for chk in range(NCH):
        g = _sc_gather_edges(table, esrc3, chk)
        delta = _edge_stage(ef2, g, chk, edge_w)
        partials.append(_sc_scatter(delta, edst3, chk))
    logit = _out_stage(partials, s_pad[:N1], subg_norm, [
        phib, phig, phibeta, owa, owb, outb, outg, outbeta,
        fc1t, fc1b, fc2t, fc2b,
    ])
    return logit


# speedup vs baseline: 2.2466x; 1.1760x over previous
import functools

import jax
import jax.numpy as jnp
from jax import lax
from jax.experimental import pallas as pl
from jax.experimental.pallas import tpu as pltpu
from jax.experimental.pallas import tpu_sc as plsc

F32 = jnp.float32

N0 = 10000
N1 = 10000
E = 320000
EDGE_IN = 16
NODE_IN = 128
H = 64
FC = 128
C = 40

NC = 2
NS = 16
NW = NC * NS

NCH = 5
ECHUNK = E // NCH
ECH = 80
EW_C = ECHUNK // NW
ENC_C = EW_C // ECH
GRP = 400
SUB = GRP // ECH
NGRP = EW_C // GRP

NPAD = 10240
SCH = 80
SNC = NPAD // (NW * SCH)

N1P = 10240
ROWS_PT = N1P // NS

BN0 = 2000
BE = 2000
BN = 2000


def _sigmoid(x):
    return 0.5 + 0.5 * jnp.tanh(0.5 * x)


def _ln_relu(x, g, b):
    m = jnp.mean(x, axis=-1, keepdims=True)
    xm = x - m
    v = jnp.mean(xm * xm, axis=-1, keepdims=True)
    return jnp.maximum(xm * lax.rsqrt(v + 1e-5) * g + b, 0.0)


def _nodepre_body(nf, hist, wa, wb, t_out, ps_out):
    pa = jnp.dot(nf[...], wa[...], preferred_element_type=F32)
    t_out[...] = jnp.concatenate([pa, hist[...]], axis=1)
    ps = jnp.dot(nf[...], wb[...], preferred_element_type=F32)
    ps_out[...] = jnp.concatenate([ps, ps], axis=1)


def _node_precompute(nf, hist, wsrc_t, wself_t):
    return pl.pallas_call(
        _nodepre_body,
        grid=(N0 // BN0,),
        in_specs=[
            pl.BlockSpec((BN0, NODE_IN), lambda i: (i, 0)),
            pl.BlockSpec((BN0, H), lambda i: (i, 0)),
            pl.BlockSpec((NODE_IN, H), lambda i: (0, 0)),
            pl.BlockSpec((NODE_IN, H), lambda i: (0, 0)),
        ],
        out_specs=[
            pl.BlockSpec((BN0, NODE_IN), lambda i: (i, 0)),
            pl.BlockSpec((BN0, NODE_IN), lambda i: (i, 0)),
        ],
        out_shape=[
            jax.ShapeDtypeStruct((N0, NODE_IN), F32),
            jax.ShapeDtypeStruct((N0, NODE_IN), F32),
        ],
    )(nf, hist, wsrc_t, wself_t)


def _gather_edges_body(chk, t_hbm, esrc_hbm, g_hbm, eidx_v, gbuf0, gbuf1,
                       gsem, wsem0, wsem1):
    c = lax.axis_index("c")
    s = lax.axis_index("s")
    w = c * NS + s

    pltpu.sync_copy(esrc_hbm.at[chk * NW + w], eidx_v)

    gbufs = (gbuf0, gbuf1)
    wsems = (wsem0, wsem1)
    wb = [None, None]
    for g in range(NGRP):
        b = g % 2
        if wb[b] is not None:
            wb[b].wait()
        cps = [
            pltpu.async_copy(t_hbm.at[eidx_v.at[g * SUB + k]],
                             gbufs[b].at[pl.ds(k * ECH, ECH)], gsem)
            for k in range(SUB)
        ]
        for cp in cps:
            cp.wait()
        wb[b] = pltpu.async_copy(
            gbufs[b], g_hbm.at[pl.ds((w * NGRP + g) * GRP, GRP)], wsems[b])
    for b in range(2):
        if wb[b] is not None:
            wb[b].wait()


def _sc_gather_edges(table, esrc3, chk):
    mesh = plsc.VectorSubcoreMesh(core_axis_name="c", subcore_axis_name="s",
                                  num_cores=NC, num_subcores=NS)
    return pl.kernel(
        functools.partial(_gather_edges_body, chk),
        out_type=jax.ShapeDtypeStruct((ECHUNK, NODE_IN), F32),
        mesh=mesh,
        scratch_types=[
            pltpu.VMEM((ENC_C, ECH), jnp.int32),
            pltpu.VMEM((GRP, NODE_IN), F32),
            pltpu.VMEM((GRP, NODE_IN), F32),
            pltpu.SemaphoreType.DMA,
            pltpu.SemaphoreType.DMA,
            pltpu.SemaphoreType.DMA,
        ],
        name=f"gather_edges_c{chk}",
    )(table, esrc3)


def _gather_self_body(ps_hbm, snid_hbm, s_hbm, sidx_v, srow_v, sem):
    c = lax.axis_index("c")
    s = lax.axis_index("s")
    w = c * NS + s

    pltpu.sync_copy(snid_hbm.at[w], sidx_v)

    def sloop(j, carry):
        pltpu.async_copy(ps_hbm.at[sidx_v.at[j]], srow_v, sem).wait()
        pltpu.sync_copy(srow_v, s_hbm.at[pl.ds((w * SNC + j) * SCH, SCH)])
        return carry

    lax.fori_loop(0, SNC, sloop, 0)


def _sc_gather_self(ps, snid3):
    mesh = plsc.VectorSubcoreMesh(core_axis_name="c", subcore_axis_name="s",
                                  num_cores=NC, num_subcores=NS)
    return pl.kernel(
        _gather_self_body,
        out_type=jax.ShapeDtypeStruct((NPAD, NODE_IN), F32),
        mesh=mesh,
        scratch_types=[
            pltpu.VMEM((SNC, SCH), jnp.int32),
            pltpu.VMEM((SCH, NODE_IN), F32),
            pltpu.SemaphoreType.DMA,
        ],
    )(ps, snid3)


def _edge_body(ef_lo, ef_hi, g_lo, g_hi,
               wih_r, wih_z, wih_n, whh_r, whh_z, whh_n, we2, mean2,
               brz_r, brz_z, bihn, bhhn, phib, phig, phibeta,
               out):
    elo = ef_lo[...]
    ehi = ef_hi[...]
    x0 = jnp.concatenate([elo[:, :EDGE_IN], ehi[:, :EDGE_IN]], axis=1)
    x1 = jnp.concatenate([elo[:, EDGE_IN:], ehi[:, EDGE_IN:]], axis=1)
    glo = g_lo[...]
    ghi = g_hi[...]
    gsrc = jnp.concatenate([glo[:, :H], ghi[:, :H]], axis=1)
    hist = jnp.concatenate([glo[:, H:], ghi[:, H:]], axis=1)

    r1 = _sigmoid(jnp.dot(x0, wih_r[...], preferred_element_type=F32)
                  + brz_r[...])
    z1 = _sigmoid(jnp.dot(x0, wih_z[...], preferred_element_type=F32)
                  + brz_z[...])
    n1 = jnp.tanh(jnp.dot(x0, wih_n[...], preferred_element_type=F32)
                  + bihn[...] + r1 * bhhn[...])
    h1 = (1.0 - z1) * n1

    r2 = _sigmoid(jnp.dot(x1, wih_r[...], preferred_element_type=F32)
                  + jnp.dot(h1, whh_r[...], preferred_element_type=F32)
                  + brz_r[...])
    z2 = _sigmoid(jnp.dot(x1, wih_z[...], preferred_element_type=F32)
                  + jnp.dot(h1, whh_z[...], preferred_element_type=F32)
                  + brz_z[...])
    n2 = jnp.tanh(jnp.dot(x1, wih_n[...], preferred_element_type=F32)
                  + bihn[...]
                  + r2 * (jnp.dot(h1, whh_n[...], preferred_element_type=F32)
                          + bhhn[...]))
    h2 = (1.0 - z2) * n2 + z2 * h1

    emb = 0.5 * (h1 + h2)
    pre = (jnp.dot(emb, we2[...], preferred_element_type=F32)
           + gsrc + phib[...])
    m = jnp.dot(pre, mean2[...], preferred_element_type=F32)
    q = jnp.dot(pre * pre, mean2[...], preferred_element_type=F32)
    v = q - m * m
    nb = jnp.maximum((pre - m) * lax.rsqrt(v + 1e-5) * phig[...]
                     + phibeta[...], 0.0)
    out[...] = jnp.maximum(nb - hist, 0.0)


def _edge_stage(ef2, g, chk, weights):
    nhb = ECHUNK // (2 * BE)
    base = chk * (ECHUNK // BE)
    wspecs = [pl.BlockSpec(w.shape, lambda i: tuple(0 for _ in w.shape))
              for w in weights]
    return pl.pallas_call(
        _edge_body,
        grid=(nhb,),
        in_specs=[
            pl.BlockSpec((BE, 2 * EDGE_IN), lambda i: (base + i, 0)),
            pl.BlockSpec((BE, 2 * EDGE_IN), lambda i: (base + nhb + i, 0)),
            pl.BlockSpec((BE, NODE_IN), lambda i: (i, 0)),
            pl.BlockSpec((BE, NODE_IN), lambda i: (nhb + i, 0)),
        ] + wspecs,
        out_specs=pl.BlockSpec((BE, NODE_IN), lambda i: (i, 0)),
        out_shape=jax.ShapeDtypeStruct((ECHUNK // 2, NODE_IN), F32),
    )(ef2, ef2, g, g, *weights)


def _scatter_body(chk, d_hbm, edst_hbm, p_hbm, acc_sh, idx_v, dbuf0, dbuf1,
                  rsem0, rsem1):
    c = lax.axis_index("c")
    s = lax.axis_index("s")

    zero16 = jnp.zeros((16,), F32)

    def zrow(r, carry):
        def zcol(k, c2):
            dbuf0[r, pl.ds(k * 16, 16)] = zero16
            return c2
        return lax.fori_loop(0, NODE_IN // 16, zcol, carry)

    lax.fori_loop(0, ECH, zrow, 0)
    for k in range(ROWS_PT // ECH):
        pltpu.sync_copy(dbuf0, acc_sh.at[pl.ds(s * ROWS_PT + k * ECH, ECH)])
    plsc.subcore_barrier()

    pltpu.sync_copy(edst_hbm.at[(chk * NC + c) * NS + s], idx_v)

    dbufs = (dbuf0, dbuf1)
    rsems = (rsem0, rsem1)
    rd = [None, None]
    rd[0] = pltpu.async_copy(d_hbm.at[pl.ds(s * EW_C, ECH)], dbufs[0],
                             rsems[0])
    for g in range(ENC_C):
        b = g % 2
        rd[b].wait()
        if g + 1 < ENC_C:
            nb = (g + 1) % 2
            rd[nb] = pltpu.async_copy(
                d_hbm.at[pl.ds(s * EW_C + (g + 1) * ECH, ECH)],
                dbufs[nb], rsems[nb])
        pltpu.sync_copy(dbufs[b], acc_sh.at[idx_v.at[g]], add=True)
    plsc.subcore_barrier()

    pltpu.sync_copy(acc_sh.at[pl.ds(s * ROWS_PT, ROWS_PT)],
                    p_hbm.at[c, pl.ds(s * ROWS_PT, ROWS_PT)])


def _sc_scatter(delta, edst3, chk):
    mesh = plsc.VectorSubcoreMesh(core_axis_name="c", subcore_axis_name="s",
                                  num_cores=NC, num_subcores=NS)
    return pl.kernel(
        functools.partial(_scatter_body, chk),
        out_type=jax.ShapeDtypeStruct((NC, N1P, NODE_IN), F32),
        mesh=mesh,
        scratch_types=[
            pltpu.VMEM_SHARED((N1P, NODE_IN), F32),
            pltpu.VMEM((ENC_C, ECH), jnp.int32),
            pltpu.VMEM((ECH, NODE_IN), F32),
            pltpu.VMEM((ECH, NODE_IN), F32),
            pltpu.SemaphoreType.DMA,
            pltpu.SemaphoreType.DMA,
        ],
        name=f"scatter_c{chk}",
    )(delta, edst3)


def _out_body(p0, p1, p2, p3, p4, sg, subg,
              phib, phig, phibeta, owa, owb, outb, outg, outbeta,
              fc1w, fc1b, fc2w, fc2b,
              out):
    delta_nb = (p0[0, :, :H] + p0[1, :, H:]
                + p1[0, :, :H] + p1[1, :, H:]
                + p2[0, :, :H] + p2[1, :, H:]
                + p3[0, :, :H] + p3[1, :, H:]
                + p4[0, :, :H] + p4[1, :, H:])
    self_h = _ln_relu(sg[:, :H] + phib[...], phig[...], phibeta[...])
    a = (delta_nb - self_h) * subg[...]
    pre = (jnp.dot(a, owa[...], preferred_element_type=F32)
           + jnp.dot(self_h, owb[...], preferred_element_type=F32)
           + outb[...])
    new_h = _ln_relu(pre, outg[...], outbeta[...])
    hf = jnp.dot(new_h, fc1w[...], preferred_element_type=F32) + fc1b[...]
    out[...] = jnp.dot(hf, fc2w[...], preferred_element_type=F32) + fc2b[...]


def _out_stage(partials, sgath, subg, weights):
    wspecs = [pl.BlockSpec(w.shape, lambda i: tuple(0 for _ in w.shape))
              for w in weights]
    pspecs = [pl.BlockSpec((NC, BN, NODE_IN), lambda i: (0, i, 0))
              for _ in partials]
    return pl.pallas_call(
        _out_body,
        grid=(N1 // BN,),
        in_specs=pspecs + [
            pl.BlockSpec((BN, NODE_IN), lambda i: (i, 0)),
            pl.BlockSpec((BN, 1), lambda i: (i, 0)),
        ] + wspecs,
        out_specs=pl.BlockSpec((BN, C), lambda i: (i, 0)),
        out_shape=jax.ShapeDtypeStruct((N1, C), F32),
    )(*partials, sgath, subg, *weights)


def kernel(node_features, edge_features, history_0, subg_norm,
           self_layer_nid, edge_src, edge_dst,
           gru_Wih, gru_Whh, gru_bih, gru_bhh,
           phi_W, phi_b, phi_g, phi_beta,
           out_W, out_b, out_g, out_beta,
           fc1_W, fc1_b, fc2_W, fc2_b):
    wsrc_t = phi_W[:, :NODE_IN].T
    wself_t = phi_W[:, H:].T
    we_t = phi_W[:, NODE_IN:].T
    wihr = gru_Wih[:H, :].T
    wihz = gru_Wih[H:2 * H, :].T
    wihn = gru_Wih[2 * H:, :].T
    whhr = gru_Whh[:H, :].T
    whhz = gru_Whh[H:2 * H, :].T
    whhn = gru_Whh[2 * H:, :].T
    brz_r = (gru_bih[:H] + gru_bhh[:H]).reshape(1, H)
    brz_z = (gru_bih[H:2 * H] + gru_bhh[H:2 * H]).reshape(1, H)
    bihn = gru_bih[2 * H:].reshape(1, H)
    bhhn = gru_bhh[2 * H:].reshape(1, H)
    phib = phi_b.reshape(1, H)
    phig = phi_g.reshape(1, H)
    phibeta = phi_beta.reshape(1, H)
    owa = out_W[:, :H].T
    owb = out_W[:, H:].T
    outb = out_b.reshape(1, H)
    outg = out_g.reshape(1, H)
    outbeta = out_beta.reshape(1, H)
    fc1t = fc1_W.T
    fc1b = fc1_b.reshape(1, FC)
    fc2t = fc2_W.T
    fc2b = fc2_b.reshape(1, C)

    def _bd(wm):
        k = wm.shape[0]
        z = jnp.zeros((k, H), F32)
        return jnp.concatenate([
            jnp.concatenate([wm, z], axis=1),
            jnp.concatenate([z, wm], axis=1),
        ], axis=0)

    def _t2(b):
        return jnp.concatenate([b, b], axis=1)

    wih_r2, wih_z2, wih_n2 = _bd(wihr), _bd(wihz), _bd(wihn)
    whh_r2, whh_z2, whh_n2 = _bd(whhr), _bd(whhz), _bd(whhn)
    we2 = _bd(we_t)
    mean2 = _bd(jnp.full((H, H), 1.0 / H, F32))

    esrc3 = edge_src.reshape(NCH * NW, ENC_C, ECH)
    edst3 = edge_dst.reshape(NCH * NW, ENC_C, ECH)
    snid_pad = jnp.concatenate(
        [self_layer_nid, jnp.zeros((NPAD - N1,), jnp.int32)]
    ).reshape(NW, SNC, SCH)
    ef2 = edge_features.reshape(E, 2 * EDGE_IN)

    edge_w = [
        wih_r2, wih_z2, wih_n2, whh_r2, whh_z2, whh_n2, we2, mean2,
        _t2(brz_r), _t2(brz_z), _t2(bihn), _t2(bhhn),
        _t2(phib), _t2(phig), _t2(phibeta),
    ]

    table, ps = _node_precompute(node_features, history_0, wsrc_t, wself_t)
    s_pad = _sc_gather_self(ps, snid_pad)
    partials = []
    for chk in range(NCH):
        g = _sc_gather_edges(table, esrc3, chk)
        delta = _edge_stage(ef2, g, chk, edge_w)
        partials.append(_sc_scatter(delta, edst3, chk))
    logit = _out_stage(partials, s_pad[:N1], subg_norm, [
        phib, phig, phibeta, owa, owb, outb, outg, outbeta,
        fc1t, fc1b, fc2t, fc2b,
    ])
    return logit
